# Initial kernel scaffold; baseline (speedup 1.0000x reference)
import jax
import jax.numpy as jnp
from jax.experimental import pallas as pl


def kernel(x, batch, v, lin):
    raise NotImplementedError("write your pallas kernel here")


# speedup vs baseline: 12.0752x; 12.0752x over previous
import functools

import jax
import jax.numpy as jnp
from jax.experimental import pallas as pl
from jax.experimental.pallas import tpu as pltpu

NUM_THETAS = 64
BUMP_STEPS = 64
NUM_SEGMENTS = 16


def _tc_body(lin_ref, x_ref, b3_ref, v_ref, out_ref, *, bn: int):
    i = pl.program_id(0)

    @pl.when(i == 0)
    def _init():
        out_ref[...] = jnp.zeros_like(out_ref)

    xb = x_ref[...]
    v = v_ref[...]
    nh = xb[:, 0:1] * v[0:1, :] + xb[:, 1:2] * v[1:2, :]
    b_row = b3_ref[0]
    seg = jax.lax.broadcasted_iota(jnp.int32, (NUM_SEGMENTS, bn), 0)
    onehot = (b_row == seg).astype(jnp.float32)
    for s in range(BUMP_STEPS):
        z = 200.0 * (lin_ref[s] - nh)
        ecc = 1.0 / (1.0 + jnp.exp(-z))
        out_ref[s] += jnp.dot(onehot, ecc, preferred_element_type=jnp.float32)


@jax.jit
def kernel(x, batch, v, lin):
    n = x.shape[0]
    bn = 2048
    nb = n // bn
    lin_flat = lin.reshape(-1).astype(jnp.float32)
    b3 = batch.astype(jnp.int32).reshape(nb, 1, bn)
    out = pl.pallas_call(
        functools.partial(_tc_body, bn=bn),
        grid=(nb,),
        in_specs=[
            pl.BlockSpec(memory_space=pltpu.SMEM),
            pl.BlockSpec((bn, 2), lambda i: (i, 0)),
            pl.BlockSpec((1, 1, bn), lambda i: (i, 0, 0)),
            pl.BlockSpec((2, NUM_THETAS), lambda i: (0, 0)),
        ],
        out_specs=pl.BlockSpec(
            (BUMP_STEPS, NUM_SEGMENTS, NUM_THETAS), lambda i: (0, 0, 0)
        ),
        out_shape=jax.ShapeDtypeStruct(
            (BUMP_STEPS, NUM_SEGMENTS, NUM_THETAS), jnp.float32
        ),
    )(lin_flat, x, b3, v)
    return jnp.moveaxis(out, 0, 1)
